# Initial kernel scaffold; baseline (speedup 1.0000x reference)
import jax
import jax.numpy as jnp
from jax.experimental import pallas as pl


def kernel(x, up_row, up_col, up_val, indices, W, b):
    raise NotImplementedError("write your pallas kernel here")


# speedup vs baseline: 9.5170x; 9.5170x over previous
import functools

import jax
import jax.numpy as jnp
from jax import lax
from jax.experimental import pallas as pl
from jax.experimental.pallas import tpu as pltpu
from jax.experimental.pallas import tpu_sc as plsc

B, N_IN, N_OUT = 4, 2500, 10000
C_IN, C_OUT, SP, NNZ = 128, 32, 9, 30000

NC, NS = 2, 16
NW = NC * NS
LANES = 16

NOP = 10240

CHUNK = 128
ACH = 15
NNZ_PAD = NS * ACH * CHUNK
ROWS_PER_TILE = NOP // NS
ZROWS = 128

WPB = NW // B
NPW = NOP // WPB
NCH = 14
GCH = (NPW + NCH - 1) // NCH
GROWS = NCH * SP
FPW = GCH * GROWS

_mesh = plsc.VectorSubcoreMesh(core_axis_name="c", subcore_axis_name="s")


@functools.partial(
    pl.kernel,
    out_type=jax.ShapeDtypeStruct((B, NOP, C_IN), jnp.float32),
    mesh=_mesh,
    scratch_types=[
        pltpu.VMEM_SHARED((NOP, C_IN), jnp.float32),
        pltpu.VMEM((ACH, CHUNK), jnp.int32),
        pltpu.VMEM((ACH, CHUNK), jnp.int32),
        pltpu.VMEM((ACH, CHUNK), jnp.float32),
        pltpu.VMEM((CHUNK, C_IN), jnp.float32),
        pltpu.VMEM((ZROWS, C_IN), jnp.float32),
    ],
)
def _pool_kernel(xf, colf, rowp, valp, pooled, shared, colv, rowv, valv, rows, zv):
    c = lax.axis_index("c")
    s = lax.axis_index("s")

    zvec = jnp.zeros((LANES,), jnp.float32)

    def zfill(i, _):
        for t in range(C_IN // LANES):
            zv[i, pl.ds(t * LANES, LANES)] = zvec
        return 0

    lax.fori_loop(0, ZROWS, zfill, 0)

    pltpu.sync_copy(rowp.at[s], rowv)
    pltpu.sync_copy(valp.at[s], valv)

    for bi in range(B // NC):
        b = c * (B // NC) + bi
        for t in range(ROWS_PER_TILE // ZROWS):
            pltpu.sync_copy(
                zv, shared.at[pl.ds(s * ROWS_PER_TILE + t * ZROWS, ZROWS)])
        plsc.subcore_barrier()

        pltpu.sync_copy(colf.at[b, s], colv)

        def chunk(j, _):
            pltpu.sync_copy(xf.at[colv.at[j]], rows)

            def scale(i16, _):
                vv = valv[j, pl.ds(i16 * LANES, LANES)]
                for u in range(LANES):
                    i = i16 * LANES + u
                    v = vv[u]
                    for t in range(C_IN // LANES):
                        sl = pl.ds(t * LANES, LANES)
                        rows[i, sl] = rows[i, sl] * v
                return 0

            lax.fori_loop(0, CHUNK // LANES, scale, 0)

            pltpu.sync_copy(rows, shared.at[rowv.at[j]], add=True)
            return 0

        lax.fori_loop(0, ACH, chunk, 0)

        plsc.subcore_barrier()
        pltpu.sync_copy(
            shared.at[pl.ds(s * ROWS_PER_TILE, ROWS_PER_TILE)],
            pooled.at[b, pl.ds(s * ROWS_PER_TILE, ROWS_PER_TILE)])
        plsc.subcore_barrier()


_BN = 2048


def _mm_body(p_ref, w_ref, z_ref):
    z_ref[...] = jnp.dot(p_ref[...], w_ref[...],
                         preferred_element_type=jnp.float32)


def _mm(pooled, w2):
    return pl.pallas_call(
        _mm_body,
        grid=(B, NOP // _BN),
        in_specs=[
            pl.BlockSpec((None, _BN, C_IN), lambda b, i: (b, i, 0)),
            pl.BlockSpec((C_IN, SP * C_OUT), lambda b, i: (0, 0)),
        ],
        out_specs=pl.BlockSpec((None, _BN, SP * C_OUT), lambda b, i: (b, i, 0)),
        out_shape=jax.ShapeDtypeStruct((B, NOP, SP * C_OUT), jnp.float32),
    )(pooled, w2)


@functools.partial(
    pl.kernel,
    out_type=jax.ShapeDtypeStruct((B, NOP, C_OUT), jnp.float32),
    mesh=_mesh,
    scratch_types=[
        pltpu.VMEM((GCH, GROWS), jnp.int32),
        pltpu.VMEM((GROWS, C_OUT), jnp.float32),
        pltpu.VMEM((GCH * NCH, C_OUT), jnp.float32),
        pltpu.VMEM((C_OUT,), jnp.float32),
    ],
    compiler_params=pltpu.CompilerParams(use_tc_tiling_on_sc=False),
)
def _spiral_kernel(zf, fidx, bias, out, fv, rv, ov, bv):
    c = lax.axis_index("c")
    s = lax.axis_index("s")
    w = s * NC + c
    b = w // WPB
    wi = w % WPB

    pltpu.sync_copy(fidx.at[b, wi], fv)
    pltpu.sync_copy(bias, bv)
    b0 = bv[pl.ds(0, LANES)]
    b1 = bv[pl.ds(LANES, LANES)]

    def chunk_body(ch, _):
        pltpu.sync_copy(zf.at[fv.at[ch]], rv)
        for n in range(NCH):
            base = n * SP
            a0 = rv[base, pl.ds(0, LANES)]
            a1 = rv[base, pl.ds(LANES, LANES)]
            for si in range(1, SP):
                a0 = a0 + rv[base + si, pl.ds(0, LANES)]
                a1 = a1 + rv[base + si, pl.ds(LANES, LANES)]
            a0 = jnp.maximum(a0 + b0, 0.0)
            a1 = jnp.maximum(a1 + b1, 0.0)
            row = ch * NCH + n
            ov[row, pl.ds(0, LANES)] = a0
            ov[row, pl.ds(LANES, LANES)] = a1
        return 0

    lax.fori_loop(0, GCH, chunk_body, 0)
    pltpu.sync_copy(ov.at[pl.ds(0, NPW)], out.at[b, pl.ds(wi * NPW, NPW)])


def kernel(x, up_row, up_col, up_val, indices, W, b):
    xf = x.reshape(B * N_IN, C_IN)
    pad = NNZ_PAD - NNZ
    rowp = jnp.concatenate(
        [up_row, jnp.zeros((pad,), jnp.int32)]).reshape(NS, ACH, CHUNK)
    colp = jnp.concatenate([up_col, jnp.zeros((pad,), jnp.int32)])
    boff = (jnp.arange(B, dtype=jnp.int32) * N_IN)[:, None]
    colf = (colp[None, :] + boff).reshape(B, NS, ACH, CHUNK)
    valp = jnp.concatenate(
        [up_val, jnp.zeros((pad,), jnp.float32)]).reshape(NS, ACH, CHUNK)

    pooled = _pool_kernel(xf, colf, rowp, valp)

    w2 = W.reshape(C_OUT, SP, C_IN).transpose(2, 1, 0).reshape(C_IN, SP * C_OUT)
    z = _mm(pooled, w2)

    indp = jnp.pad(indices, ((0, NOP - N_OUT), (0, 0)))
    fid = (indp * SP
           + jnp.arange(SP, dtype=jnp.int32)[None, :]).reshape(-1)
    fid = fid.reshape(WPB, NPW * SP)
    fid = jnp.pad(fid, ((0, 0), (0, FPW - NPW * SP)))
    zoff = (jnp.arange(B, dtype=jnp.int32) * (NOP * SP))[:, None, None]
    fidx = (fid[None, :, :] + zoff).reshape(B, WPB, GCH, GROWS)

    zflat = z.reshape(B * NOP * SP, C_OUT)
    return _spiral_kernel(zflat, fidx, b)[:, :N_OUT, :]


# speedup vs baseline: 10.7500x; 1.1296x over previous
import functools

import jax
import jax.numpy as jnp
from jax import lax
from jax.experimental import pallas as pl
from jax.experimental.pallas import tpu as pltpu
from jax.experimental.pallas import tpu_sc as plsc

B, N_IN, N_OUT = 4, 2500, 10000
C_IN, C_OUT, SP, NNZ = 128, 32, 9, 30000

NC, NS = 2, 16
NW = NC * NS
LANES = 16

NOP = 10240

CHUNK = 48
ACH = 40
NNZ_PAD = NS * ACH * CHUNK
ROWS_PER_TILE = NOP // NS
ZROWS = 8
N_IN_PAD = 2560

WPB = NW // B
NPW = NOP // WPB
NCH = 14
GCH = (NPW + NCH - 1) // NCH
GROWS = NCH * SP
FPW = GCH * GROWS

_mesh = plsc.VectorSubcoreMesh(core_axis_name="c", subcore_axis_name="s")


@functools.partial(
    pl.kernel,
    out_type=jax.ShapeDtypeStruct((B, NOP, C_IN), jnp.float32),
    mesh=_mesh,
    scratch_types=[
        pltpu.VMEM_SHARED((NOP, C_IN), jnp.float32),
        pltpu.VMEM((ACH, CHUNK), jnp.int32),
        pltpu.VMEM((ACH, CHUNK), jnp.int32),
        pltpu.VMEM((ACH, CHUNK), jnp.float32),
        pltpu.VMEM((2, CHUNK, C_IN), jnp.float32),
        pltpu.VMEM((2, CHUNK, C_IN), jnp.float32),
        pltpu.VMEM((ZROWS, C_IN), jnp.float32),
        pltpu.SemaphoreType.DMA,
        pltpu.SemaphoreType.DMA,
        pltpu.SemaphoreType.DMA,
        pltpu.SemaphoreType.DMA,
    ],
)
def _pool_kernel(xf, colf, rowp, valp, pooled, shared, colv, rowv, valv,
                 grows, srows, zv, gs0, gs1, ss0, ss1):
    c = lax.axis_index("c")
    s = lax.axis_index("s")
    gsems = (gs0, gs1)
    ssems = (ss0, ss1)

    zvec = jnp.zeros((LANES,), jnp.float32)

    def zfill(i, _):
        for t in range(C_IN // LANES):
            zv[i, pl.ds(t * LANES, LANES)] = zvec
        return 0

    lax.fori_loop(0, ZROWS, zfill, 0)

    pltpu.sync_copy(colf.at[s], colv)
    pltpu.sync_copy(rowp.at[s], rowv)
    pltpu.sync_copy(valp.at[s], valv)

    for bi in range(B // NC):
        b = c * (B // NC) + bi
        xb = xf.at[pl.ds(b * N_IN_PAD, N_IN_PAD)]
        for t in range(ROWS_PER_TILE // ZROWS):
            pltpu.sync_copy(
                zv, shared.at[pl.ds(s * ROWS_PER_TILE + t * ZROWS, ZROWS)])
        plsc.subcore_barrier()

        pltpu.async_copy(xb.at[colv.at[0]], grows.at[0], gs0)
        pltpu.async_copy(xb.at[colv.at[1]], grows.at[1], gs1)

        def pair(g, _):
            for t2 in range(2):
                j = g * 2 + t2
                gb = grows.at[t2]
                sb = srows.at[t2]
                pltpu.make_async_copy(xb.at[colv.at[j]], gb, gsems[t2]).wait()
                @pl.when(g >= 1)
                def _():
                    pltpu.make_async_copy(
                        sb, shared.at[rowv.at[j]], ssems[t2]).wait()

                def scale(i16, _):
                    vv = valv[j, pl.ds(i16 * LANES, LANES)]
                    for u in range(LANES):
                        i = i16 * LANES + u
                        v = vv[u]
                        for t in range(C_IN // LANES):
                            sl = pl.ds(t * LANES, LANES)
                            sb[i, sl] = gb[i, sl] * v
                    return 0

                lax.fori_loop(0, CHUNK // LANES, scale, 0)

                pltpu.async_copy(sb, shared.at[rowv.at[j]], ssems[t2],
                                 add=True)

                @pl.when(g < ACH // 2 - 1)
                def _():
                    pltpu.async_copy(xb.at[colv.at[j + 2]], gb, gsems[t2])
            return 0

        lax.fori_loop(0, ACH // 2, pair, 0)

        for t2 in range(2):
            pltpu.make_async_copy(
                srows.at[t2], shared.at[rowv.at[ACH - 2 + t2]],
                ssems[t2]).wait()

        plsc.subcore_barrier()
        pltpu.sync_copy(
            shared.at[pl.ds(s * ROWS_PER_TILE, ROWS_PER_TILE)],
            pooled.at[b, pl.ds(s * ROWS_PER_TILE, ROWS_PER_TILE)])
        plsc.subcore_barrier()


_BN = 4096


_QR = _BN // 4
_ZR = _BN * SP * C_OUT // 128


def _mm_body(p_ref, w_ref, z_ref):
    for s in range(SP):
        acc = jnp.dot(p_ref[:, 0, :], w_ref[0, :, pl.ds(s * 128, 128)],
                      preferred_element_type=jnp.float32)
        for a in range(1, 4):
            acc = acc + jnp.dot(p_ref[:, a, :],
                                w_ref[a, :, pl.ds(s * 128, 128)],
                                preferred_element_type=jnp.float32)
        z_ref[pl.ds(s * _QR, _QR)] = acc.astype(jnp.bfloat16)


def _mm(pooled, w4):
    pooled3 = pooled.reshape(B * NOP // 4, 4, C_IN)
    return pl.pallas_call(
        _mm_body,
        grid=(B * NOP // _BN,),
        in_specs=[
            pl.BlockSpec((_QR, 4, C_IN), lambda i: (i, 0, 0)),
            pl.BlockSpec((4, C_IN, SP * 128), lambda i: (0, 0, 0)),
        ],
        out_specs=pl.BlockSpec((_ZR, 128), lambda i: (i, 0)),
        out_shape=jax.ShapeDtypeStruct((B * NOP * SP * C_OUT // 128, 128),
                                       jnp.bfloat16),
    )(pooled3, w4)


@functools.partial(
    pl.kernel,
    out_type=jax.ShapeDtypeStruct((B, NOP, C_OUT), jnp.float32),
    mesh=_mesh,
    scratch_types=[
        pltpu.VMEM((GCH, GROWS), jnp.int32),
        pltpu.VMEM((4, GROWS, C_OUT), jnp.bfloat16),
        pltpu.VMEM((GCH * NCH, C_OUT), jnp.float32),
        pltpu.VMEM((C_OUT,), jnp.float32),
        pltpu.SemaphoreType.DMA,
        pltpu.SemaphoreType.DMA,
        pltpu.SemaphoreType.DMA,
        pltpu.SemaphoreType.DMA,
    ],
    compiler_params=pltpu.CompilerParams(use_tc_tiling_on_sc=False,
                                         needs_layout_passes=False),
)
def _spiral_kernel(zf, fidx, bias, out, fv, rv, ov, bv, gs0, gs1, gs2, gs3):
    c = lax.axis_index("c")
    s = lax.axis_index("s")
    w = s * NC + c
    b = w // WPB
    wi = w % WPB
    gsems = (gs0, gs1, gs2, gs3)

    pltpu.sync_copy(fidx.at[b, wi], fv)
    pltpu.sync_copy(bias, bv)
    b0 = bv[pl.ds(0, LANES)]
    b1 = bv[pl.ds(LANES, LANES)]

    for t2 in range(4):
        pltpu.async_copy(zf.at[fv.at[t2]], rv.at[t2], gsems[t2])

    def quad_body(g, _):
        for t2 in range(4):
            ch = g * 4 + t2
            buf = rv.at[t2]
            pltpu.make_async_copy(zf.at[fv.at[ch]], buf, gsems[t2]).wait()
            for n in range(NCH):
                base = n * SP
                a0, a1 = plsc.unpack(buf[base, pl.ds(0, 2 * LANES)],
                                     format=plsc.PackFormat.INTERLEAVED)
                for si in range(1, SP):
                    e, o = plsc.unpack(buf[base + si, pl.ds(0, 2 * LANES)],
                                       format=plsc.PackFormat.INTERLEAVED)
                    a0 = a0 + e
                    a1 = a1 + o
                a0 = jnp.maximum(a0 + b0, 0.0)
                a1 = jnp.maximum(a1 + b1, 0.0)
                row = ch * NCH + n
                ov[row, pl.ds(0, LANES)] = a0
                ov[row, pl.ds(LANES, LANES)] = a1

            @pl.when(g < GCH // 4 - 1)
            def _():
                pltpu.async_copy(zf.at[fv.at[ch + 4]], buf, gsems[t2])
        return 0

    lax.fori_loop(0, GCH // 4, quad_body, 0)
    pltpu.sync_copy(ov.at[pl.ds(0, NPW)], out.at[b, pl.ds(wi * NPW, NPW)])


def kernel(x, up_row, up_col, up_val, indices, W, b):
    xf = jnp.pad(x, ((0, 0), (0, N_IN_PAD - N_IN), (0, 0))).reshape(
        B * N_IN_PAD, C_IN)
    pad = NNZ_PAD - NNZ
    rowp = jnp.concatenate(
        [up_row, jnp.zeros((pad,), jnp.int32)]).reshape(NS, ACH, CHUNK)
    colf = jnp.concatenate(
        [up_col, jnp.zeros((pad,), jnp.int32)]).reshape(NS, ACH, CHUNK)
    valp = jnp.concatenate(
        [up_val, jnp.zeros((pad,), jnp.float32)]).reshape(NS, ACH, CHUNK)

    pooled = _pool_kernel(xf, colf, rowp, valp)

    wk = W.reshape(C_OUT, SP, C_IN).transpose(2, 1, 0)
    w4 = jnp.stack([
        jnp.pad(wk, ((0, 0), (0, 0), (32 * a, 96 - 32 * a))).reshape(
            C_IN, SP * 128)
        for a in range(4)])
    zq = _mm(pooled, w4)

    indp = jnp.pad(indices, ((0, NOP - N_OUT), (0, 0)))
    g = indp[None, :, :] + (jnp.arange(B, dtype=jnp.int32) * NOP)[:, None, None]
    s_arr = jnp.arange(SP, dtype=jnp.int32)[None, None, :]
    fid = (g // _BN) * (_BN * SP) + s_arr * _BN + g % _BN
    fid = fid.reshape(B, WPB, NPW * SP)
    fid = jnp.pad(fid, ((0, 0), (0, 0), (0, FPW - NPW * SP)))
    fidx = fid.reshape(B, WPB, GCH, GROWS)

    zflat = zq.reshape(B * NOP * SP, C_OUT)
    b_eo = jnp.concatenate([b[0::2], b[1::2]])
    op = _spiral_kernel(zflat, fidx, b_eo)
    out = op.reshape(B, NOP, 2, LANES).transpose(0, 1, 3, 2).reshape(
        B, NOP, C_OUT)
    return out[:, :N_OUT, :]


# speedup vs baseline: 12.2804x; 1.1424x over previous
import functools

import jax
import jax.numpy as jnp
from jax import lax
from jax.experimental import pallas as pl
from jax.experimental.pallas import tpu as pltpu
from jax.experimental.pallas import tpu_sc as plsc

B, N_IN, N_OUT = 4, 2500, 10000
C_IN, C_OUT, SP, NNZ = 128, 32, 9, 30000

NC, NS = 2, 16
NW = NC * NS
LANES = 16

NOP = 10240

CHUNK = 48
ACH = 40
NNZ_PAD = NS * ACH * CHUNK
ROWS_PER_TILE = NOP // NS
ZROWS = 8
N_IN_PAD = 2560

WPB = NW // B
NPW = NOP // WPB
NCH = 14
GCH = (NPW + NCH - 1) // NCH
GROWS = NCH * SP
FPW = GCH * GROWS

_mesh = plsc.VectorSubcoreMesh(core_axis_name="c", subcore_axis_name="s")


@functools.partial(
    pl.kernel,
    out_type=jax.ShapeDtypeStruct((B, NOP, C_IN), jnp.float32),
    mesh=_mesh,
    scratch_types=[
        pltpu.VMEM_SHARED((NOP, C_IN), jnp.float32),
        pltpu.VMEM((ACH, CHUNK), jnp.int32),
        pltpu.VMEM((ACH, CHUNK), jnp.int32),
        pltpu.VMEM((ACH, CHUNK), jnp.float32),
        pltpu.VMEM((2, CHUNK, C_IN), jnp.float32),
        pltpu.VMEM((2, CHUNK, C_IN), jnp.float32),
        pltpu.VMEM((ZROWS, C_IN), jnp.float32),
        pltpu.SemaphoreType.DMA,
        pltpu.SemaphoreType.DMA,
        pltpu.SemaphoreType.DMA,
        pltpu.SemaphoreType.DMA,
    ],
)
def _pool_kernel(xf, colf, rowp, valp, pooled, shared, colv, rowv, valv,
                 grows, srows, zv, gs0, gs1, ss0, ss1):
    c = lax.axis_index("c")
    s = lax.axis_index("s")
    gsems = (gs0, gs1)
    ssems = (ss0, ss1)

    zvec = jnp.zeros((LANES,), jnp.float32)

    def zfill(i, _):
        for t in range(C_IN // LANES):
            zv[i, pl.ds(t * LANES, LANES)] = zvec
        return 0

    lax.fori_loop(0, ZROWS, zfill, 0)

    pltpu.sync_copy(colf.at[s], colv)
    pltpu.sync_copy(rowp.at[s], rowv)
    pltpu.sync_copy(valp.at[s], valv)

    for bi in range(B // NC):
        b = c * (B // NC) + bi
        xb = xf.at[pl.ds(b * N_IN_PAD, N_IN_PAD)]
        for t in range(ROWS_PER_TILE // ZROWS):
            pltpu.sync_copy(
                zv, shared.at[pl.ds(s * ROWS_PER_TILE + t * ZROWS, ZROWS)])
        plsc.subcore_barrier()

        pltpu.async_copy(xb.at[colv.at[0]], grows.at[0], gs0)
        pltpu.async_copy(xb.at[colv.at[1]], grows.at[1], gs1)

        def pair(g, _):
            for t2 in range(2):
                j = g * 2 + t2
                gb = grows.at[t2]
                sb = srows.at[t2]
                pltpu.make_async_copy(xb.at[colv.at[j]], gb, gsems[t2]).wait()
                @pl.when(g >= 1)
                def _():
                    pltpu.make_async_copy(
                        sb, shared.at[rowv.at[j]], ssems[t2]).wait()

                def scale(i16, _):
                    vv = valv[j, pl.ds(i16 * LANES, LANES)]
                    for u in range(LANES):
                        i = i16 * LANES + u
                        v = vv[u]
                        for t in range(C_IN // LANES):
                            sl = pl.ds(t * LANES, LANES)
                            sb[i, sl] = gb[i, sl] * v
                    return 0

                lax.fori_loop(0, CHUNK // LANES, scale, 0)

                pltpu.async_copy(sb, shared.at[rowv.at[j]], ssems[t2],
                                 add=True)

                @pl.when(g < ACH // 2 - 1)
                def _():
                    pltpu.async_copy(xb.at[colv.at[j + 2]], gb, gsems[t2])
            return 0

        lax.fori_loop(0, ACH // 2, pair, 0)

        for t2 in range(2):
            pltpu.make_async_copy(
                srows.at[t2], shared.at[rowv.at[ACH - 2 + t2]],
                ssems[t2]).wait()

        plsc.subcore_barrier()
        pltpu.sync_copy(
            shared.at[pl.ds(s * ROWS_PER_TILE, ROWS_PER_TILE)],
            pooled.at[b, pl.ds(s * ROWS_PER_TILE, ROWS_PER_TILE)])
        plsc.subcore_barrier()


_BN = 4096


_QR = _BN // 4
_ZR = _BN * SP * C_OUT // 128


def _mm_body(p_ref, w_ref, z_ref):
    for s in range(SP):
        acc = jnp.dot(p_ref[:, 0, :], w_ref[0, :, pl.ds(s * 128, 128)],
                      preferred_element_type=jnp.float32)
        for a in range(1, 4):
            acc = acc + jnp.dot(p_ref[:, a, :],
                                w_ref[a, :, pl.ds(s * 128, 128)],
                                preferred_element_type=jnp.float32)
        z_ref[pl.ds(s * _QR, _QR)] = acc


def _mm(pooled, w4):
    pooled3 = pooled.reshape(B * NOP // 4, 4, C_IN)
    return pl.pallas_call(
        _mm_body,
        grid=(B * NOP // _BN,),
        in_specs=[
            pl.BlockSpec((_QR, 4, C_IN), lambda i: (i, 0, 0)),
            pl.BlockSpec((4, C_IN, SP * 128), lambda i: (0, 0, 0)),
        ],
        out_specs=pl.BlockSpec((_ZR, 128), lambda i: (i, 0)),
        out_shape=jax.ShapeDtypeStruct((B * NOP * SP * C_OUT // 128, 128),
                                       jnp.float32),
    )(pooled3, w4)


@functools.partial(
    pl.kernel,
    out_type=jax.ShapeDtypeStruct((B, NOP, C_OUT), jnp.float32),
    mesh=_mesh,
    scratch_types=[
        pltpu.VMEM((GCH, GROWS), jnp.int32),
        pltpu.VMEM((4, GROWS, C_OUT), jnp.float32),
        pltpu.VMEM((GCH * NCH, C_OUT), jnp.float32),
        pltpu.VMEM((C_OUT,), jnp.float32),
        pltpu.SemaphoreType.DMA,
        pltpu.SemaphoreType.DMA,
        pltpu.SemaphoreType.DMA,
        pltpu.SemaphoreType.DMA,
    ],
    compiler_params=pltpu.CompilerParams(use_tc_tiling_on_sc=False),
)
def _spiral_kernel(zf, fidx, bias, out, fv, rv, ov, bv, gs0, gs1, gs2, gs3):
    c = lax.axis_index("c")
    s = lax.axis_index("s")
    w = s * NC + c
    b = w // WPB
    wi = w % WPB
    gsems = (gs0, gs1, gs2, gs3)

    pltpu.sync_copy(fidx.at[b, wi], fv)
    pltpu.sync_copy(bias, bv)
    b0 = bv[pl.ds(0, LANES)]
    b1 = bv[pl.ds(LANES, LANES)]

    for t2 in range(4):
        pltpu.async_copy(zf.at[fv.at[t2]], rv.at[t2], gsems[t2])

    def quad_body(g, _):
        for t2 in range(4):
            ch = g * 4 + t2
            buf = rv.at[t2]
            pltpu.make_async_copy(zf.at[fv.at[ch]], buf, gsems[t2]).wait()
            for n in range(NCH):
                base = n * SP
                a0 = buf[base, pl.ds(0, LANES)]
                a1 = buf[base, pl.ds(LANES, LANES)]
                for si in range(1, SP):
                    a0 = a0 + buf[base + si, pl.ds(0, LANES)]
                    a1 = a1 + buf[base + si, pl.ds(LANES, LANES)]
                a0 = jnp.maximum(a0 + b0, 0.0)
                a1 = jnp.maximum(a1 + b1, 0.0)
                row = ch * NCH + n
                ov[row, pl.ds(0, LANES)] = a0
                ov[row, pl.ds(LANES, LANES)] = a1

            @pl.when(g < GCH // 4 - 1)
            def _():
                pltpu.async_copy(zf.at[fv.at[ch + 4]], buf, gsems[t2])
        return 0

    lax.fori_loop(0, GCH // 4, quad_body, 0)
    pltpu.sync_copy(ov.at[pl.ds(0, NPW)], out.at[b, pl.ds(wi * NPW, NPW)])


def kernel(x, up_row, up_col, up_val, indices, W, b):
    xf = jnp.pad(x, ((0, 0), (0, N_IN_PAD - N_IN), (0, 0))).reshape(
        B * N_IN_PAD, C_IN)
    pad = NNZ_PAD - NNZ
    rowp = jnp.concatenate(
        [up_row, jnp.zeros((pad,), jnp.int32)]).reshape(NS, ACH, CHUNK)
    colf = jnp.concatenate(
        [up_col, jnp.zeros((pad,), jnp.int32)]).reshape(NS, ACH, CHUNK)
    valp = jnp.concatenate(
        [up_val, jnp.zeros((pad,), jnp.float32)]).reshape(NS, ACH, CHUNK)

    pooled = _pool_kernel(xf, colf, rowp, valp)

    wk = W.reshape(C_OUT, SP, C_IN).transpose(2, 1, 0)
    w4 = jnp.stack([
        jnp.pad(wk, ((0, 0), (0, 0), (32 * a, 96 - 32 * a))).reshape(
            C_IN, SP * 128)
        for a in range(4)])
    zq = _mm(pooled, w4)

    indp = jnp.pad(indices, ((0, NOP - N_OUT), (0, 0)))
    g = indp[None, :, :] + (jnp.arange(B, dtype=jnp.int32) * NOP)[:, None, None]
    s_arr = jnp.arange(SP, dtype=jnp.int32)[None, None, :]
    fid = (g // _BN) * (_BN * SP) + s_arr * _BN + g % _BN
    fid = fid.reshape(B, WPB, NPW * SP)
    fid = jnp.pad(fid, ((0, 0), (0, 0), (0, FPW - NPW * SP)))
    fidx = fid.reshape(B, WPB, GCH, GROWS)

    zflat = zq.reshape(B * NOP * SP, C_OUT)
    return _spiral_kernel(zflat, fidx, b)[:, :N_OUT, :]
